# baseline (speedup 1.0000x reference)
import jax
import jax.numpy as jnp
from jax import lax
from jax.experimental import pallas as pl
from jax.experimental.pallas import tpu as pltpu
from jax.experimental.pallas import tpu_sc as plsc

_N_STATES = 1000000
_N_ACT = 16
_B = 16384
_NC, _NS = 2, 16
_NW = _NC * _NS
_BPW = _B // _NW
_CHUNK = 16
_NCHUNK = _BPW // _CHUNK
_TAIL0 = (_N_STATES // 128) * 128
_CLAMP = _TAIL0 - 128

_LN2 = 0.6931471805599453


def _log_1_16(s):
    bits = plsc.bitcast(s, jnp.int32)
    e = (bits >> 23) - 127
    mant = plsc.bitcast((bits & 0x007FFFFF) | 0x3F800000, jnp.float32)
    t = (mant - 1.0) / (mant + 1.0)
    u = t * t
    logm = 2.0 * t * (1.0 + u * (1.0 / 3.0 + u * (0.2 + u * (1.0 / 7.0))))
    return e.astype(jnp.float32) * _LN2 + logm


def _body(policy_t, tail_hbm, feat_hbm, act_hbm, out_hbm, idx_v, act_v, buf,
          tail_v, out_v, sem):
    wid = lax.axis_index("s") * _NC + lax.axis_index("c")
    base = wid * _BPW

    pltpu.sync_copy(feat_hbm.at[pl.ds(base, _BPW)], idx_v)
    pltpu.sync_copy(act_hbm.at[pl.ds(base, _BPW)], act_v)
    pltpu.sync_copy(tail_hbm, tail_v)

    lanes = lax.iota(jnp.int32, _N_ACT)

    def fire(c, slot0):
        svec0 = idx_v[pl.ds(c * _CHUNK, _CHUNK)]
        for i in range(_CHUNK):
            s = svec0[i]
            c0 = pl.multiple_of(
                jnp.minimum((s >> 7) << 7, _CLAMP).astype(jnp.int32), 128)
            for r in range(2):
                pltpu.async_copy(
                    policy_t.at[pl.ds(r * 8, 8), pl.ds(c0, 128)],
                    buf.at[slot0 + i, pl.ds(r * 8, 8)], sem)

    def drain(slot0):
        for i in range(_CHUNK):
            for r in range(2):
                pltpu.make_async_copy(
                    policy_t.at[pl.ds(r * 8, 8), pl.ds(0, 128)],
                    buf.at[slot0 + i, pl.ds(r * 8, 8)], sem).wait()

    fire(jnp.int32(0), jnp.int32(0))
    fire(jnp.int32(1), jnp.int32(_CHUNK))

    def chunk(k, carry):
        p = jnp.remainder(k, 3) * _CHUNK
        drain(p)
        fire(jnp.minimum(k + 2, _NCHUNK - 1),
             jnp.remainder(k + 2, 3) * _CHUNK)

        sl = pl.ds(k * _CHUNK, _CHUNK)
        svec = idx_v[sl]
        mvec = svec & 127
        istail = svec >= _TAIL0
        tidx = jnp.where(istail, svec - _TAIL0, 0)
        slots = p + lanes

        def logits(avec):
            main = plsc.load_gather(buf, [slots, avec, mvec])
            tail = plsc.load_gather(tail_v, [tidx, avec])
            return jnp.where(istail, tail, main)

        cols = [logits(jnp.full((16,), a, jnp.int32)) for a in range(_N_ACT)]
        m = cols[0]
        for a in range(1, _N_ACT):
            m = jnp.maximum(m, cols[a])
        ssum = jnp.exp(cols[0] - m)
        for a in range(1, _N_ACT):
            ssum = ssum + jnp.exp(cols[a] - m)
        sel = logits(act_v[sl])
        out_v[sl] = sel - m - _log_1_16(ssum)
        return carry

    lax.fori_loop(0, _NCHUNK, chunk, None)
    drain(jnp.int32(jnp.remainder(_NCHUNK, 3) * _CHUNK))
    drain(jnp.int32(jnp.remainder(_NCHUNK + 1, 3) * _CHUNK))
    pltpu.sync_copy(out_v, out_hbm.at[pl.ds(base, _BPW)])


_sc_call_cache = []


def _sc_call():
    if not _sc_call_cache:
        _sc_call_cache.append(pl.kernel(
            _body,
            out_type=jax.ShapeDtypeStruct((_B,), jnp.float32),
            mesh=plsc.VectorSubcoreMesh(core_axis_name="c",
                                        subcore_axis_name="s",
                                        num_cores=_NC, num_subcores=_NS),
            compiler_params=pltpu.CompilerParams(
                needs_layout_passes=False,
            ),
            scratch_types=[
                pltpu.VMEM((_BPW,), jnp.int32),
                pltpu.VMEM((_BPW,), jnp.int32),
                pltpu.VMEM((3 * _CHUNK, _N_ACT, 128), jnp.float32),
                pltpu.VMEM((_N_STATES - _TAIL0, _N_ACT), jnp.float32),
                pltpu.VMEM((_BPW,), jnp.float32),
                pltpu.SemaphoreType.DMA,
            ],
        ))
    return _sc_call_cache[0]


def kernel(policy, feat, taken_actions):
    tail = lax.slice(policy, (_TAIL0, 0), (_N_STATES, _N_ACT))
    return _sc_call()(policy.T, tail, feat.astype(jnp.int32),
                      taken_actions.astype(jnp.int32))


# speedup vs baseline: 1.0150x; 1.0150x over previous
import jax
import jax.numpy as jnp
from jax import lax
from jax.experimental import pallas as pl
from jax.experimental.pallas import tpu as pltpu
from jax.experimental.pallas import tpu_sc as plsc

_N_STATES = 1000000
_N_ACT = 16
_B = 16384
_NC, _NS = 2, 16
_NW = _NC * _NS
_BPW = _B // _NW
_CHUNK = 16
_NCHUNK = _BPW // _CHUNK
_TAIL0 = (_N_STATES // 128) * 128
_CLAMP = _TAIL0 - 128

_LN2 = 0.6931471805599453


def _log_1_16(s):
    bits = plsc.bitcast(s, jnp.int32)
    e = (bits >> 23) - 127
    mant = plsc.bitcast((bits & 0x007FFFFF) | 0x3F800000, jnp.float32)
    t = (mant - 1.0) / (mant + 1.0)
    u = t * t
    logm = 2.0 * t * (1.0 + u * (1.0 / 3.0 + u * (0.2 + u * (1.0 / 7.0))))
    return e.astype(jnp.float32) * _LN2 + logm


def _body(policy_t, tail_hbm, feat_hbm, act_hbm, out_hbm, idx_v, act_v, buf,
          tail_v, out_v, sem):
    wid = lax.axis_index("s") * _NC + lax.axis_index("c")
    base = wid * _BPW

    pltpu.sync_copy(feat_hbm.at[pl.ds(base, _BPW)], idx_v)
    pltpu.sync_copy(act_hbm.at[pl.ds(base, _BPW)], act_v)
    pltpu.sync_copy(tail_hbm, tail_v)

    lanes = lax.iota(jnp.int32, _N_ACT)

    def fire(c, slot0):
        svec0 = idx_v[pl.ds(c * _CHUNK, _CHUNK)]
        for i in range(_CHUNK):
            s = svec0[i]
            c0 = pl.multiple_of(
                jnp.minimum((s >> 7) << 7, _CLAMP).astype(jnp.int32), 128)
            pltpu.async_copy(policy_t.at[:, pl.ds(c0, 128)],
                             buf.at[slot0 + i], sem)

    def drain(slot0):
        for i in range(_CHUNK):
            pltpu.make_async_copy(policy_t.at[:, pl.ds(0, 128)],
                                  buf.at[slot0 + i], sem).wait()

    fire(jnp.int32(0), jnp.int32(0))
    fire(jnp.int32(1), jnp.int32(_CHUNK))

    def chunk(k, carry):
        p = jnp.remainder(k, 3) * _CHUNK
        drain(p)
        fire(jnp.minimum(k + 2, _NCHUNK - 1),
             jnp.remainder(k + 2, 3) * _CHUNK)

        sl = pl.ds(k * _CHUNK, _CHUNK)
        svec = idx_v[sl]
        mvec = svec & 127
        istail = svec >= _TAIL0
        tidx = jnp.where(istail, svec - _TAIL0, 0)
        slots = p + lanes

        def logits(avec):
            main = plsc.load_gather(buf, [slots, avec, mvec])
            tail = plsc.load_gather(tail_v, [tidx, avec])
            return jnp.where(istail, tail, main)

        cols = [logits(jnp.full((16,), a, jnp.int32)) for a in range(_N_ACT)]
        m = cols[0]
        for a in range(1, _N_ACT):
            m = jnp.maximum(m, cols[a])
        ssum = jnp.exp(cols[0] - m)
        for a in range(1, _N_ACT):
            ssum = ssum + jnp.exp(cols[a] - m)
        sel = logits(act_v[sl])
        out_v[sl] = sel - m - _log_1_16(ssum)
        return carry

    lax.fori_loop(0, _NCHUNK, chunk, None)
    drain(jnp.int32(jnp.remainder(_NCHUNK, 3) * _CHUNK))
    drain(jnp.int32(jnp.remainder(_NCHUNK + 1, 3) * _CHUNK))
    pltpu.sync_copy(out_v, out_hbm.at[pl.ds(base, _BPW)])


_sc_call_cache = []


def _sc_call():
    if not _sc_call_cache:
        _sc_call_cache.append(pl.kernel(
            _body,
            out_type=jax.ShapeDtypeStruct((_B,), jnp.float32),
            mesh=plsc.VectorSubcoreMesh(core_axis_name="c",
                                        subcore_axis_name="s",
                                        num_cores=_NC, num_subcores=_NS),
            compiler_params=pltpu.CompilerParams(
                needs_layout_passes=False,
            ),
            scratch_types=[
                pltpu.VMEM((_BPW,), jnp.int32),
                pltpu.VMEM((_BPW,), jnp.int32),
                pltpu.VMEM((3 * _CHUNK, _N_ACT, 128), jnp.float32),
                pltpu.VMEM((_N_STATES - _TAIL0, _N_ACT), jnp.float32),
                pltpu.VMEM((_BPW,), jnp.float32),
                pltpu.SemaphoreType.DMA,
            ],
        ))
    return _sc_call_cache[0]


def kernel(policy, feat, taken_actions):
    tail = lax.slice(policy, (_TAIL0, 0), (_N_STATES, _N_ACT))
    return _sc_call()(policy.T, tail, feat.astype(jnp.int32),
                      taken_actions.astype(jnp.int32))


# speedup vs baseline: 1.0597x; 1.0441x over previous
import jax
import jax.numpy as jnp
from jax import lax
from jax.experimental import pallas as pl
from jax.experimental.pallas import tpu as pltpu
from jax.experimental.pallas import tpu_sc as plsc

_N_STATES = 1000000
_N_ACT = 16
_B = 16384
_NC, _NS = 2, 16
_NW = _NC * _NS
_BPW = _B // _NW
_CHUNK = 16
_NCHUNK = _BPW // _CHUNK
_TAIL0 = (_N_STATES // 128) * 128
_CLAMP = _TAIL0 - 128

_LN2 = 0.6931471805599453


def _log_1_16(s):
    bits = plsc.bitcast(s, jnp.int32)
    e = (bits >> 23) - 127
    mant = plsc.bitcast((bits & 0x007FFFFF) | 0x3F800000, jnp.float32)
    t = (mant - 1.0) / (mant + 1.0)
    u = t * t
    logm = 2.0 * t * (1.0 + u * (1.0 / 3.0 + u * (0.2 + u * (1.0 / 7.0))))
    return e.astype(jnp.float32) * _LN2 + logm


def _body(policy_t, tail_hbm, feat_hbm, act_hbm, out_hbm, idx_v, act_v, buf,
          tail_v, out_v, sem):
    wid = lax.axis_index("s") * _NC + lax.axis_index("c")
    base = wid * _BPW

    pltpu.sync_copy(feat_hbm.at[pl.ds(base, _BPW)], idx_v)
    pltpu.sync_copy(act_hbm.at[pl.ds(base, _BPW)], act_v)
    pltpu.sync_copy(tail_hbm, tail_v)

    lanes = lax.iota(jnp.int32, _N_ACT)

    def fire(c, slot0):
        svec0 = idx_v[pl.ds(c * _CHUNK, _CHUNK)]
        for i in range(_CHUNK):
            s = svec0[i]
            c0 = pl.multiple_of(
                jnp.minimum((s >> 7) << 7, _CLAMP).astype(jnp.int32), 128)
            pltpu.async_copy(policy_t.at[:, pl.ds(c0, 128)],
                             buf.at[slot0 + i], sem)

    def drain(slot0):
        for i in range(_CHUNK):
            pltpu.make_async_copy(policy_t.at[:, pl.ds(0, 128)],
                                  buf.at[slot0 + i], sem).wait()

    fire(jnp.int32(0), jnp.int32(0))
    fire(jnp.int32(1), jnp.int32(_CHUNK))

    def chunk(k, carry):
        p = jnp.remainder(k, 3) * _CHUNK
        drain(p)

        @pl.when(k + 2 < _NCHUNK)
        def _():
            fire(k + 2, jnp.remainder(k + 2, 3) * _CHUNK)

        sl = pl.ds(k * _CHUNK, _CHUNK)
        svec = idx_v[sl]
        mvec = svec & 127
        istail = svec >= _TAIL0
        tidx = jnp.where(istail, svec - _TAIL0, 0)
        slots = p + lanes

        def logits(avec):
            main = plsc.load_gather(buf, [slots, avec, mvec])
            tail = plsc.load_gather(tail_v, [tidx, avec])
            return jnp.where(istail, tail, main)

        cols = [logits(jnp.full((16,), a, jnp.int32)) for a in range(_N_ACT)]
        m = cols[0]
        for a in range(1, _N_ACT):
            m = jnp.maximum(m, cols[a])
        ssum = jnp.exp(cols[0] - m)
        for a in range(1, _N_ACT):
            ssum = ssum + jnp.exp(cols[a] - m)
        sel = logits(act_v[sl])
        out_v[sl] = sel - m - _log_1_16(ssum)
        return carry

    lax.fori_loop(0, _NCHUNK, chunk, None)
    pltpu.sync_copy(out_v, out_hbm.at[pl.ds(base, _BPW)])


_sc_call_cache = []


def _sc_call():
    if not _sc_call_cache:
        _sc_call_cache.append(pl.kernel(
            _body,
            out_type=jax.ShapeDtypeStruct((_B,), jnp.float32),
            mesh=plsc.VectorSubcoreMesh(core_axis_name="c",
                                        subcore_axis_name="s",
                                        num_cores=_NC, num_subcores=_NS),
            compiler_params=pltpu.CompilerParams(
                needs_layout_passes=False,
            ),
            scratch_types=[
                pltpu.VMEM((_BPW,), jnp.int32),
                pltpu.VMEM((_BPW,), jnp.int32),
                pltpu.VMEM((3 * _CHUNK, _N_ACT, 128), jnp.float32),
                pltpu.VMEM((_N_STATES - _TAIL0, _N_ACT), jnp.float32),
                pltpu.VMEM((_BPW,), jnp.float32),
                pltpu.SemaphoreType.DMA,
            ],
        ))
    return _sc_call_cache[0]


def kernel(policy, feat, taken_actions):
    tail = lax.slice(policy, (_TAIL0, 0), (_N_STATES, _N_ACT))
    return _sc_call()(policy.T, tail, feat.astype(jnp.int32),
                      taken_actions.astype(jnp.int32))
